# baseline (speedup 1.0000x reference)
import functools

import jax
import jax.numpy as jnp
from jax import lax
from jax.experimental import pallas as pl
from jax.experimental.pallas import tpu as pltpu
from jax.experimental.pallas import tpu_sc as plsc

_CHUNKS = 12


def _sc_block(x_flat, codes_p, table, B, C, S, NC, NS):
    NP = 56
    NW = NC * NS
    PB = B // NW
    XW = C * S
    OW = 2 * XW
    mesh = plsc.VectorSubcoreMesh(core_axis_name="c", subcore_axis_name="s")

    chunk_plan = [(16 * k, 4 * k) for k in range(_CHUNKS)] + [(S - 16, (S - 16) // 4)]

    @functools.partial(
        pl.kernel,
        out_type=jax.ShapeDtypeStruct((B * OW,), jnp.float32),
        mesh=mesh,
        compiler_params=pltpu.CompilerParams(needs_layout_passes=False),
        scratch_types=[
            pltpu.VMEM((PB * NP,), jnp.int32),
            pltpu.VMEM((NP, C), jnp.float32),
            pltpu.VMEM((NP, C), jnp.float32),
            pltpu.VMEM((XW,), jnp.float32),
            pltpu.VMEM((XW,), jnp.float32),
            pltpu.VMEM_SHARED((NS * 2 * XW,), jnp.float32),
            pltpu.SemaphoreType.DMA,
            pltpu.SemaphoreType.DMA,
            pltpu.SemaphoreType.DMA,
            pltpu.SemaphoreType.DMA,
            pltpu.SemaphoreType.DMA,
            pltpu.SemaphoreType.DMA,
            pltpu.SemaphoreType.DMA,
            pltpu.SemaphoreType.DMA,
        ],
    )
    def body(x_hbm, idx_hbm, table_hbm, out_hbm, idx_v, emb_a, emb_b,
             u_a, u_b, x_sp,
             gsem0, gsem1, usem0, usem1, xisem0, xisem1, xosem0, xosem1):
        sid = lax.axis_index("s")
        wid = sid * NC + lax.axis_index("c")
        b0 = wid * PB
        embs = (emb_a, emb_b)
        us = (u_a, u_b)
        xbase = sid * (2 * XW)
        gsems = (gsem0, gsem1)
        usems = (usem0, usem1)
        xisems = (xisem0, xisem1)
        xosems = (xosem0, xosem1)
        pltpu.sync_copy(idx_hbm.at[pl.ds(wid * PB * NP, PB * NP)], idx_v)

        lanes = lax.iota(jnp.int32, 16)
        j4 = lax.shift_right_logical(lanes, 2)
        bases = [j4 + n0 for (_, n0) in chunk_plan]

        def issue_gather(bi, slot):
            pltpu.async_copy(table_hbm.at[idx_v.at[pl.ds(bi * NP, NP)]],
                             embs[slot], gsems[slot])

        def wait_gather(bi, slot):
            pltpu.make_async_copy(table_hbm.at[idx_v.at[pl.ds(bi * NP, NP)]],
                                  embs[slot], gsems[slot]).wait()

        def xslot(slot):
            return x_sp.at[pl.ds(xbase + slot * XW, XW)]

        def issue_xin(bi, slot):
            pltpu.async_copy(x_hbm.at[pl.ds((b0 + bi) * XW, XW)], xslot(slot),
                             xisems[slot])

        def wait_xin(bi, slot):
            pltpu.make_async_copy(x_hbm.at[pl.ds((b0 + bi) * XW, XW)],
                                  xslot(slot), xisems[slot]).wait()

        def issue_xout(bi, slot):
            pltpu.async_copy(xslot(slot),
                             out_hbm.at[pl.ds((b0 + bi) * OW, XW)],
                             xosems[slot])

        def wait_xout(bi, slot):
            pltpu.make_async_copy(xslot(slot),
                                  out_hbm.at[pl.ds((b0 + bi) * OW, XW)],
                                  xosems[slot]).wait()

        def issue_u(bi, slot):
            pltpu.async_copy(us[slot],
                             out_hbm.at[pl.ds((b0 + bi) * OW + XW, XW)],
                             usems[slot])

        def wait_u(bi, slot):
            pltpu.make_async_copy(us[slot],
                                  out_hbm.at[pl.ds((b0 + bi) * OW + XW, XW)],
                                  usems[slot]).wait()

        def upsample(slot):
            ev = embs[slot]
            uv = us[slot]

            def cloop(c, carry):
                cvec = jnp.full((16,), c, jnp.int32)
                crow = c * S
                for k, (s0, _) in enumerate(chunk_plan):
                    vals = plsc.load_gather(ev, [bases[k], cvec])
                    uv[pl.ds(crow + s0, 16)] = vals
                return carry

            lax.fori_loop(0, C, cloop, 0)

        issue_gather(0, 0)
        issue_xin(0, 0)
        issue_gather(1, 1)
        issue_xin(1, 1)

        def half(bp, bi, slot):
            wait_xin(bi, slot)
            issue_xout(bi, slot)
            wait_gather(bi, slot)

            @pl.when(bp >= 1)
            def _():
                wait_u(bi, slot)

            upsample(slot)
            issue_u(bi, slot)

            @pl.when(bi + 2 < PB)
            def _():
                wait_xout(bi, slot)
                issue_gather(bi + 2, slot)
                issue_xin(bi + 2, slot)

        def pair(bp, carry):
            half(bp, 2 * bp, 0)
            half(bp, 2 * bp + 1, 1)
            return carry

        lax.fori_loop(0, PB // 2, pair, 0)

        wait_u(PB - 2, 0)
        wait_u(PB - 1, 1)
        wait_xout(PB - 2, 0)
        wait_xout(PB - 1, 1)

    return body(x_flat, codes_p, table)


def kernel(x, codes, W_emb):
    b, c, S = x.shape
    _, N = codes.shape

    info = plsc.get_sparse_core_info()
    NC, NS = info.num_cores, info.num_subcores
    NW = NC * NS

    NP = 56
    codes_p = jnp.pad(codes.astype(jnp.int32), ((0, 0), (0, NP - N)))

    out_flat = _sc_block(x.reshape(-1), codes_p.reshape(-1), W_emb,
                         b, c, S, NC, NS)
    return out_flat.reshape(b, 2 * c, S)


# speedup vs baseline: 1.0017x; 1.0017x over previous
import functools

import jax
import jax.numpy as jnp
from jax import lax
from jax.experimental import pallas as pl
from jax.experimental.pallas import tpu as pltpu
from jax.experimental.pallas import tpu_sc as plsc

_CHUNKS = 12


def _sc_block(x_flat, codes_p, table, B, C, S, NC, NS):
    NP = 56
    NW = NC * NS
    PB = B // NW
    XW = C * S
    OW = 2 * XW
    mesh = plsc.VectorSubcoreMesh(core_axis_name="c", subcore_axis_name="s")

    chunk_plan = [(16 * k, 4 * k) for k in range(_CHUNKS)] + [(S - 16, (S - 16) // 4)]

    @functools.partial(
        pl.kernel,
        out_type=jax.ShapeDtypeStruct((B * OW,), jnp.float32),
        mesh=mesh,
        compiler_params=pltpu.CompilerParams(needs_layout_passes=False),
        scratch_types=[
            pltpu.VMEM((PB * NP,), jnp.int32),
            pltpu.VMEM((NP, C), jnp.float32),
            pltpu.VMEM((NP, C), jnp.float32),
            pltpu.VMEM((XW,), jnp.float32),
            pltpu.VMEM((XW,), jnp.float32),
            pltpu.VMEM_SHARED((NS * 2 * XW,), jnp.float32),
            pltpu.SemaphoreType.DMA,
            pltpu.SemaphoreType.DMA,
            pltpu.SemaphoreType.DMA,
            pltpu.SemaphoreType.DMA,
            pltpu.SemaphoreType.DMA,
            pltpu.SemaphoreType.DMA,
            pltpu.SemaphoreType.DMA,
            pltpu.SemaphoreType.DMA,
        ],
    )
    def body(x_hbm, idx_hbm, table_hbm, out_hbm, idx_v, emb_a, emb_b,
             u_a, u_b, x_sp,
             gsem0, gsem1, usem0, usem1, xisem0, xisem1, xosem0, xosem1):
        sid = lax.axis_index("s")
        wid = sid * NC + lax.axis_index("c")
        b0 = wid * PB
        embs = (emb_a, emb_b)
        us = (u_a, u_b)
        xbase = sid * (2 * XW)
        gsems = (gsem0, gsem1)
        usems = (usem0, usem1)
        xisems = (xisem0, xisem1)
        xosems = (xosem0, xosem1)
        pltpu.sync_copy(idx_hbm.at[pl.ds(wid * PB * NP, PB * NP)], idx_v)

        lanes = lax.iota(jnp.int32, 16)
        j4 = lax.shift_right_logical(lanes, 2)
        bases = [j4 + n0 for (_, n0) in chunk_plan]

        def issue_gather(bi, slot):
            pltpu.async_copy(table_hbm.at[idx_v.at[pl.ds(bi * NP, NP)]],
                             embs[slot], gsems[slot])

        def wait_gather(bi, slot):
            pltpu.make_async_copy(table_hbm.at[idx_v.at[pl.ds(bi * NP, NP)]],
                                  embs[slot], gsems[slot]).wait()

        def xslot(slot):
            return x_sp.at[pl.ds(xbase + slot * XW, XW)]

        def issue_xin(bi, slot):
            pltpu.async_copy(x_hbm.at[pl.ds((b0 + bi) * XW, XW)], xslot(slot),
                             xisems[slot])

        def wait_xin(bi, slot):
            pltpu.make_async_copy(x_hbm.at[pl.ds((b0 + bi) * XW, XW)],
                                  xslot(slot), xisems[slot]).wait()

        def issue_xout(bi, slot):
            pltpu.async_copy(xslot(slot),
                             out_hbm.at[pl.ds((b0 + bi) * OW, XW)],
                             xosems[slot])

        def wait_xout(bi, slot):
            pltpu.make_async_copy(xslot(slot),
                                  out_hbm.at[pl.ds((b0 + bi) * OW, XW)],
                                  xosems[slot]).wait()

        def issue_u(bi, slot):
            pltpu.async_copy(us[slot],
                             out_hbm.at[pl.ds((b0 + bi) * OW + XW, XW)],
                             usems[slot])

        def wait_u(bi, slot):
            pltpu.make_async_copy(us[slot],
                                  out_hbm.at[pl.ds((b0 + bi) * OW + XW, XW)],
                                  usems[slot]).wait()

        def upsample(slot):
            ev = embs[slot]
            uv = us[slot]

            def cloop(c, carry):
                cvec = jnp.full((16,), c, jnp.int32)
                crow = c * S
                for k, (s0, _) in enumerate(chunk_plan):
                    vals = plsc.load_gather(ev, [bases[k], cvec])
                    uv[pl.ds(crow + s0, 16)] = vals
                return carry

            lax.fori_loop(0, C, cloop, 0)

        issue_gather(0, 0)
        issue_xin(0, 0)
        issue_gather(1, 1)
        issue_xin(1, 1)

        def half(bp, bi, slot):
            with jax.named_scope("wait_xin"):
                wait_xin(bi, slot)
            issue_xout(bi, slot)
            with jax.named_scope("wait_gather"):
                wait_gather(bi, slot)

            @pl.when(bp >= 1)
            def _():
                with jax.named_scope("wait_u"):
                    wait_u(bi, slot)

            with jax.named_scope("upsample"):
                upsample(slot)
            issue_u(bi, slot)

            @pl.when(bi + 2 < PB)
            def _():
                with jax.named_scope("wait_xout"):
                    wait_xout(bi, slot)
                issue_gather(bi + 2, slot)
                issue_xin(bi + 2, slot)

        def pair(bp, carry):
            half(bp, 2 * bp, 0)
            half(bp, 2 * bp + 1, 1)
            return carry

        lax.fori_loop(0, PB // 2, pair, 0)

        wait_u(PB - 2, 0)
        wait_u(PB - 1, 1)
        wait_xout(PB - 2, 0)
        wait_xout(PB - 1, 1)

    return body(x_flat, codes_p, table)


def kernel(x, codes, W_emb):
    b, c, S = x.shape
    _, N = codes.shape

    info = plsc.get_sparse_core_info()
    NC, NS = info.num_cores, info.num_subcores
    NW = NC * NS

    NP = 56
    codes_p = jnp.pad(codes.astype(jnp.int32), ((0, 0), (0, NP - N)))

    out_flat = _sc_block(x.reshape(-1), codes_p.reshape(-1), W_emb,
                         b, c, S, NC, NS)
    return out_flat.reshape(b, 2 * c, S)


# speedup vs baseline: 2.9766x; 2.9716x over previous
import functools

import numpy as np
import jax
import jax.numpy as jnp
from jax import lax
from jax.experimental import pallas as pl
from jax.experimental.pallas import tpu as pltpu
from jax.experimental.pallas import tpu_sc as plsc


def _sc_gather(table, idx_grouped, B, D, NC, NS):
    NW, nchunk, ch = idx_grouped.shape
    mesh = plsc.VectorSubcoreMesh(core_axis_name="c", subcore_axis_name="s")

    @functools.partial(
        pl.kernel,
        out_type=jax.ShapeDtypeStruct((B, D), jnp.float32),
        mesh=mesh,
        scratch_types=[
            pltpu.VMEM((nchunk, ch), jnp.int32),
            pltpu.VMEM((ch, D), jnp.float32),
            pltpu.SemaphoreType.DMA,
        ],
    )
    def gather(table_hbm, idx_hbm, out_hbm, idx_v, rows_v, sem):
        wid = lax.axis_index("s") * NC + lax.axis_index("c")
        pltpu.sync_copy(idx_hbm.at[wid], idx_v)
        base = wid * (nchunk * ch)

        def body(k, _):
            pltpu.async_copy(table_hbm.at[idx_v.at[k]], rows_v, sem).wait()
            pltpu.sync_copy(rows_v, out_hbm.at[pl.ds(base + k * ch, ch)])
            return _

        lax.fori_loop(0, nchunk, body, None)

    return gather(table, idx_grouped)


def _fuse(x, emb2d, G, N, BB):
    b, c, S = x.shape

    def body(x_ref, emb_ref, g_ref, out_ref):
        out_ref[:, :c, :] = x_ref[...]
        for j in range(BB):
            out_ref[j, c:, :] = lax.dot_general(
                emb_ref[pl.ds(j * N, N), :],
                g_ref[...],
                (((0,), (0,)), ((), ())),
                preferred_element_type=jnp.float32,
                precision=lax.Precision.HIGHEST,
            )

    return pl.pallas_call(
        body,
        grid=(b // BB,),
        in_specs=[
            pl.BlockSpec((BB, c, S), lambda i: (i, 0, 0)),
            pl.BlockSpec((BB * N, c), lambda i: (i, 0)),
            pl.BlockSpec((N, S), lambda i: (0, 0)),
        ],
        out_specs=pl.BlockSpec((BB, 2 * c, S), lambda i: (i, 0, 0)),
        out_shape=jax.ShapeDtypeStruct((b, 2 * c, S), jnp.float32),
    )(x, emb2d, G)


def kernel(x, codes, W_emb):
    b, c, S = x.shape
    _, N = codes.shape
    V, D = W_emb.shape
    B = b * N

    info = plsc.get_sparse_core_info()
    NC, NS = info.num_cores, info.num_subcores
    NW = NC * NS
    per_w = B // NW
    ch = 64
    nchunk = per_w // ch

    idx = codes.reshape(NW, nchunk, ch).astype(jnp.int32)
    emb = _sc_gather(W_emb, idx, B, D, NC, NS)

    src = np.floor(np.arange(S) * (N / S)).astype(np.int32)
    G = jnp.asarray((src[None, :] == np.arange(N)[:, None]).astype(np.float32))

    return _fuse(x, emb, G, N, BB=16)


# speedup vs baseline: 3.0898x; 1.0380x over previous
import functools

import numpy as np
import jax
import jax.numpy as jnp
from jax import lax
from jax.experimental import pallas as pl
from jax.experimental.pallas import tpu as pltpu
from jax.experimental.pallas import tpu_sc as plsc


def _sc_gather(table, idx_grouped, B, D, NC, NS):
    NW, nchunk, ch = idx_grouped.shape
    mesh = plsc.VectorSubcoreMesh(core_axis_name="c", subcore_axis_name="s")

    @functools.partial(
        pl.kernel,
        out_type=jax.ShapeDtypeStruct((B, D), jnp.float32),
        mesh=mesh,
        scratch_types=[
            pltpu.VMEM((nchunk, ch), jnp.int32),
            pltpu.VMEM((ch, D), jnp.float32),
            pltpu.SemaphoreType.DMA,
        ],
    )
    def gather(table_hbm, idx_hbm, out_hbm, idx_v, rows_v, sem):
        wid = lax.axis_index("s") * NC + lax.axis_index("c")
        pltpu.sync_copy(idx_hbm.at[wid], idx_v)
        base = wid * (nchunk * ch)

        def body(k, _):
            pltpu.async_copy(table_hbm.at[idx_v.at[k]], rows_v, sem).wait()
            pltpu.sync_copy(rows_v, out_hbm.at[pl.ds(base + k * ch, ch)])
            return _

        lax.fori_loop(0, nchunk, body, None)

    return gather(table, idx_grouped)


def _fuse(x, emb2d, G, N, BB):
    b, c, S = x.shape

    def body(x_ref, emb_ref, g_ref, out_ref):
        out_ref[:, :c, :] = x_ref[...]
        for j in range(BB):
            out_ref[j, c:, :] = lax.dot_general(
                emb_ref[pl.ds(j * N, N), :],
                g_ref[...],
                (((0,), (0,)), ((), ())),
                preferred_element_type=jnp.float32,
                precision=lax.Precision.HIGHEST,
            )

    return pl.pallas_call(
        body,
        grid=(b // BB,),
        in_specs=[
            pl.BlockSpec((BB, c, S), lambda i: (i, 0, 0)),
            pl.BlockSpec((BB * N, c), lambda i: (i, 0)),
            pl.BlockSpec((N, S), lambda i: (0, 0)),
        ],
        out_specs=pl.BlockSpec((BB, 2 * c, S), lambda i: (i, 0, 0)),
        out_shape=jax.ShapeDtypeStruct((b, 2 * c, S), jnp.float32),
    )(x, emb2d, G)


def kernel(x, codes, W_emb):
    b, c, S = x.shape
    _, N = codes.shape
    V, D = W_emb.shape
    B = b * N

    info = plsc.get_sparse_core_info()
    NC, NS = info.num_cores, info.num_subcores
    NW = NC * NS
    per_w = B // NW
    ch = 64
    nchunk = per_w // ch

    idx = codes.reshape(NW, nchunk, ch).astype(jnp.int32)
    emb = _sc_gather(W_emb, idx, B, D, NC, NS)

    src = np.floor(np.arange(S) * (N / S)).astype(np.int32)
    G = jnp.asarray((src[None, :] == np.arange(N)[:, None]).astype(np.float32))

    return _fuse(x, emb, G, N, BB=32)
